# baseline (speedup 1.0000x reference)
import functools

import jax
import jax.numpy as jnp
from jax import lax
from jax.experimental import pallas as pl
from jax.experimental.pallas import tpu as pltpu
from jax.experimental.pallas import tpu_sc as plsc

N = 10000
NP = 10240
E = 160000
FB, FC = 2, 256
D = FB * FC

NW = 32
EC = 2000
NCH = E // EC
G = 8
NBUF = 18

NPT = NP // NW
BLK = 2048
CAP = 80 * BLK
SCH = 80
NRND = 4

_mesh = plsc.VectorSubcoreMesh(core_axis_name="c", subcore_axis_name="s")
_sc_params = pltpu.CompilerParams(needs_layout_passes=False)


def _wid():
    return lax.axis_index("s") * 2 + lax.axis_index("c")


@functools.partial(
    pl.kernel,
    mesh=_mesh,
    out_type=(
        jax.ShapeDtypeStruct((NP,), jnp.float32),
        jax.ShapeDtypeStruct((NP,), jnp.float32),
    ),
    scratch_types=[
        pltpu.VMEM((EC,), jnp.int32),
        pltpu.VMEM((EC,), jnp.int32),
        pltpu.VMEM((NPT,), jnp.float32),
        pltpu.VMEM((NPT,), jnp.float32),
    ],
    compiler_params=_sc_params,
)
def _degrees(src_hbm, dst_hbm, dego_hbm, degi_hbm, sbuf, dbuf, acco, acci):
    wid = _wid()
    lo = wid * NPT
    hi = lo + NPT
    zeros = jnp.zeros((16,), jnp.float32)
    ones = jnp.ones((16,), jnp.float32)
    for j in range(NPT // 16):
        acco[pl.ds(j * 16, 16)] = zeros
        acci[pl.ds(j * 16, 16)] = zeros

    def chunk_body(ec, _):
        off = pl.multiple_of(ec * EC, 8)
        pltpu.sync_copy(src_hbm.at[pl.ds(off, EC)], sbuf)
        pltpu.sync_copy(dst_hbm.at[pl.ds(off, EC)], dbuf)

        def grp(i, _):
            svec = sbuf[pl.ds(i * 16, 16)]
            dvec = dbuf[pl.ds(i * 16, 16)]
            ms = (svec >= lo) & (svec < hi)
            md = (dvec >= lo) & (dvec < hi)
            si = jnp.clip(svec - lo, 0, NPT - 1)
            di = jnp.clip(dvec - lo, 0, NPT - 1)
            plsc.addupdate_scatter(acco, [si], ones, mask=ms)
            plsc.addupdate_scatter(acci, [di], ones, mask=md)
            return 0

        return lax.fori_loop(0, EC // 16, grp, 0)

    lax.fori_loop(0, NCH, chunk_body, 0)
    pltpu.sync_copy(acco, dego_hbm.at[pl.ds(lo, NPT)])
    pltpu.sync_copy(acci, degi_hbm.at[pl.ds(lo, NPT)])


@functools.partial(
    pl.kernel,
    mesh=_mesh,
    out_type=(
        jax.ShapeDtypeStruct((NW, CAP), jnp.int32),
        jax.ShapeDtypeStruct((NW, CAP), jnp.int32),
        jax.ShapeDtypeStruct((NW * 16,), jnp.int32),
    ),
    scratch_types=[
        pltpu.VMEM((EC,), jnp.int32),
        pltpu.VMEM((EC,), jnp.int32),
        pltpu.VMEM((2 * BLK + 16,), jnp.int32),
        pltpu.VMEM((2 * BLK + 16,), jnp.int32),
        pltpu.VMEM((16,), jnp.int32),
    ],
    compiler_params=_sc_params,
)
def _bucket(src_hbm, dst_hbm, bsrc_hbm, bldst_hbm, cnt_hbm,
            sbuf, dbuf, psrc, pdst, stg):
    wid = _wid()
    lo = pl.multiple_of(wid * NPT, 8)
    hi = lo + NPT

    def chunk_body(ec, carry):
        off = pl.multiple_of(ec * EC, 8)
        pltpu.sync_copy(src_hbm.at[pl.ds(off, EC)], sbuf)
        pltpu.sync_copy(dst_hbm.at[pl.ds(off, EC)], dbuf)

        def grp(i, ptr):
            for u in range(5):
                off16 = (i * 5 + u) * 16
                svec = sbuf[pl.ds(off16, 16)]
                dvec = dbuf[pl.ds(off16, 16)]
                m = (dvec >= lo) & (dvec < hi)
                plsc.store_compressed(psrc.at[pl.ds(ptr, 16)], svec, mask=m)
                plsc.store_compressed(pdst.at[pl.ds(ptr, 16)], dvec - lo,
                                      mask=m)
                ptr = ptr + plsc.all_reduce_population_count(m)[0]
            return ptr

        ptr = lax.fori_loop(0, EC // 80, grp, carry[0])

        def do_flush(c):
            p, b = c
            boff = pl.multiple_of(b * BLK, 8)
            pltpu.sync_copy(psrc.at[pl.ds(0, BLK)],
                            bsrc_hbm.at[wid, pl.ds(boff, BLK)])
            pltpu.sync_copy(pdst.at[pl.ds(0, BLK)],
                            bldst_hbm.at[wid, pl.ds(boff, BLK)])
            for j in range(BLK // 16):
                psrc[pl.ds(j * 16, 16)] = psrc[pl.ds(BLK + j * 16, 16)]
                pdst[pl.ds(j * 16, 16)] = pdst[pl.ds(BLK + j * 16, 16)]
            return (p - BLK, b + 1)

        return lax.cond(ptr >= BLK, do_flush, lambda c: c, (ptr, carry[1]))

    ptr, blk = lax.fori_loop(0, NCH, chunk_body,
                             (jnp.int32(0), jnp.int32(0)))
    psrc[pl.ds(ptr, 16)] = jnp.zeros((16,), jnp.int32)
    pdst[pl.ds(ptr, 16)] = jnp.full((16,), NPT, jnp.int32)
    boff = pl.multiple_of(blk * BLK, 8)
    pltpu.sync_copy(psrc.at[pl.ds(0, BLK)], bsrc_hbm.at[wid, pl.ds(boff, BLK)])
    pltpu.sync_copy(pdst.at[pl.ds(0, BLK)], bldst_hbm.at[wid, pl.ds(boff, BLK)])
    stg[...] = jnp.broadcast_to(blk * BLK + ptr, (16,))
    pltpu.sync_copy(stg, cnt_hbm.at[pl.ds(pl.multiple_of(wid * 16, 8), 16)])


@functools.partial(
    pl.kernel,
    mesh=_mesh,
    out_type=jax.ShapeDtypeStruct((NP, D), jnp.float32),
    scratch_types=[
        pltpu.VMEM((BLK,), jnp.int32),
        pltpu.VMEM((BLK,), jnp.int32),
        pltpu.VMEM((BLK + 32,), jnp.int32),
        pltpu.VMEM((BLK + 32,), jnp.int32),
        pltpu.VMEM((NBUF, G, D), jnp.float32),
        pltpu.VMEM((SCH + 1, D), jnp.float32),
    ] + [pltpu.SemaphoreType.DMA] * NBUF,
    compiler_params=_sc_params,
)
def _scatter_rows(bsrc_hbm, bldst_hbm, cnt_hbm, z_hbm, agg_hbm,
                  lbs, lbd, psrc, pdst, gbuf, acc, *sems):
    wid = _wid()
    zeros = jnp.zeros((16,), jnp.float32)
    trash = jnp.full((16,), SCH, jnp.int32)
    zsrc = jnp.zeros((16,), jnp.int32)

    pltpu.sync_copy(cnt_hbm.at[pl.ds(pl.multiple_of(wid * 16, 8), 16)],
                    lbs.at[pl.ds(0, 16)])
    cnt = lbs[pl.ds(0, 16)][0]
    cnt16 = ((cnt + 15) // 16) * 16
    nblk = (cnt16 + BLK - 1) // BLK

    def round_body(rnd, _):
        rlo = rnd * SCH
        rhi = rlo + SCH

        def zrow(r, _):
            for k in range(D // 16):
                acc[r, pl.ds(k * 16, 16)] = zeros
            return 0

        lax.fori_loop(0, SCH + 1, zrow, 0)

        def blk_body(bi, _):
            boff = pl.multiple_of(bi * BLK, 8)
            pltpu.sync_copy(bsrc_hbm.at[wid, pl.ds(boff, BLK)], lbs)
            pltpu.sync_copy(bldst_hbm.at[wid, pl.ds(boff, BLK)], lbd)
            ngrp = jnp.minimum(BLK, cnt16 - bi * BLK) // 16

            def grp(i, ptr):
                svec = lbs[pl.ds(i * 16, 16)]
                dvec = lbd[pl.ds(i * 16, 16)]
                m = (dvec >= rlo) & (dvec < rhi)
                plsc.store_compressed(psrc.at[pl.ds(ptr, 16)], svec, mask=m)
                plsc.store_compressed(pdst.at[pl.ds(ptr, 16)], dvec - rlo,
                                      mask=m)
                return ptr + plsc.all_reduce_population_count(m)[0]

            ptr = lax.fori_loop(0, ngrp, grp, jnp.int32(0))
            psrc[pl.ds(ptr, 16)] = zsrc
            pdst[pl.ds(ptr, 16)] = trash
            nb = (ptr + G - 1) // G

            def fire(g, buf):
                goff = pl.multiple_of(g * G, 8)
                pltpu.async_copy(z_hbm.at[psrc.at[pl.ds(goff, G)]],
                                 gbuf.at[buf], sems[buf])

            def wait(g, buf):
                goff = pl.multiple_of(g * G, 8)
                pltpu.make_async_copy(z_hbm.at[psrc.at[pl.ds(goff, G)]],
                                      gbuf.at[buf], sems[buf]).wait()

            def accum(g, buf):
                goff = pl.multiple_of(g * G, 8)

                def rows4(jj, _):
                    dvq = pdst[pl.ds(goff + jj * 4, 16)]
                    for q in range(4):
                        d = dvq[q]
                        gg = jj * 4 + q
                        for k in range(D // 16):
                            plsc.addupdate(
                                acc.at[d, pl.ds(k * 16, 16)],
                                gbuf[buf, gg, pl.ds(k * 16, 16)],
                            )
                    return 0

                lax.fori_loop(0, G // 4, rows4, 0)

            for u in range(NBUF):
                @pl.when(u < nb)
                def _(u=u):
                    fire(u, u)

            def drainN(j, _):
                g0 = j * NBUF
                for u in range(NBUF):
                    @pl.when(g0 + u < nb)
                    def _(u=u):
                        wait(g0 + u, u)
                        accum(g0 + u, u)

                        @pl.when(g0 + u + NBUF < nb)
                        def _(u=u):
                            fire(g0 + u + NBUF, u)

                return 0

            lax.fori_loop(0, (nb + NBUF - 1) // NBUF, drainN, 0)
            return 0

        lax.fori_loop(0, nblk, blk_body, 0)
        out_lo = pl.multiple_of(wid * NPT + rlo, 8)
        pltpu.sync_copy(acc.at[pl.ds(0, SCH)], agg_hbm.at[pl.ds(out_lo, SCH)])
        return 0

    lax.fori_loop(0, NRND, round_body, 0)


TN = 512


def _norm(deg):
    return lax.rsqrt(jnp.maximum(deg, 1.0))


def _l1_body(x_ref, dego_ref, w1_ref, z_ref):
    xb = x_ref[0]
    ns = _norm(dego_ref[...])
    y = lax.dot_general(xb, w1_ref[...], (((0,), (0,)), ((), ())),
                        preferred_element_type=jnp.float32)
    z_ref[...] = y * ns[:, None]


def _layer1_matmul(x_p, deg_out, W1):
    return pl.pallas_call(
        _l1_body,
        grid=(NP // TN, FB),
        in_specs=[
            pl.BlockSpec((1, FC, TN), lambda n, b: (b, 0, n)),
            pl.BlockSpec((TN,), lambda n, b: (n,)),
            pl.BlockSpec((FC, FC), lambda n, b: (0, 0)),
        ],
        out_specs=pl.BlockSpec((TN, FC), lambda n, b: (n, b)),
        out_shape=jax.ShapeDtypeStruct((NP, D), jnp.float32),
    )(x_p, deg_out, W1)


def _l2_body(agg_ref, dego_ref, degi_ref, b1_ref, w2_ref, z_ref):
    ns = _norm(dego_ref[...])
    nd = _norm(degi_ref[...])
    h = jnp.maximum(agg_ref[...] * nd[:, None] + b1_ref[...][None, :], 0.0)
    z_ref[...] = lax.dot_general(
        h * ns[:, None], w2_ref[...], (((1,), (0,)), ((), ())),
        preferred_element_type=jnp.float32)


def _layer2_matmul(agg1, deg_out, deg_in, b1, W2):
    return pl.pallas_call(
        _l2_body,
        grid=(NP // TN, FB),
        in_specs=[
            pl.BlockSpec((TN, FC), lambda n, b: (n, b)),
            pl.BlockSpec((TN,), lambda n, b: (n,)),
            pl.BlockSpec((TN,), lambda n, b: (n,)),
            pl.BlockSpec((FC,), lambda n, b: (0,)),
            pl.BlockSpec((FC, FC), lambda n, b: (0, 0)),
        ],
        out_specs=pl.BlockSpec((TN, FC), lambda n, b: (n, b)),
        out_shape=jax.ShapeDtypeStruct((NP, D), jnp.float32),
    )(agg1, deg_out, deg_in, b1, W2)


def _fin_body(agg_ref, degi_ref, b2_ref, o_ref):
    nd = _norm(degi_ref[...])
    o_ref[...] = agg_ref[...] * nd[:, None] + b2_ref[...][None, :]


def _finalize(agg2, deg_in, b2):
    return pl.pallas_call(
        _fin_body,
        grid=(NP // TN, FB),
        in_specs=[
            pl.BlockSpec((TN, FC), lambda n, b: (n, b)),
            pl.BlockSpec((TN,), lambda n, b: (n,)),
            pl.BlockSpec((FC,), lambda n, b: (0,)),
        ],
        out_specs=pl.BlockSpec((TN, FC), lambda n, b: (n, b)),
        out_shape=jax.ShapeDtypeStruct((NP, D), jnp.float32),
    )(agg2, deg_in, b2)


def kernel(inputs, edge_index, W1, b1, W2, b2):
    b, c, h, w = inputs.shape
    x = inputs.reshape(b, c, h * w)
    x_p = jnp.pad(x, ((0, 0), (0, 0), (0, NP - N)))
    src = edge_index[0]
    dst = edge_index[1]

    deg_out, deg_in = _degrees(src, dst)
    bsrc, bldst, bcnt = _bucket(src, dst)
    z1 = _layer1_matmul(x_p, deg_out, W1)
    agg1 = _scatter_rows(bsrc, bldst, bcnt, z1)
    z2 = _layer2_matmul(agg1, deg_out, deg_in, b1, W2)
    agg2 = _scatter_rows(bsrc, bldst, bcnt, z2)
    onb = _finalize(agg2, deg_in, b2)

    out = onb[:N].reshape(N, b, c).transpose(1, 2, 0)
    return out.reshape(b, c, h, w)


# speedup vs baseline: 1.0594x; 1.0594x over previous
import functools

import jax
import jax.numpy as jnp
from jax import lax
from jax.experimental import pallas as pl
from jax.experimental.pallas import tpu as pltpu
from jax.experimental.pallas import tpu_sc as plsc

N = 10000
NP = 10240
E = 160000
FB, FC = 2, 256
D = FB * FC

NW = 32
EC = 2000
NCH = E // EC
G = 16
NBUF = 11

NPT = NP // NW
BLK = 2048
CAP = 80 * BLK
SCH = 40
NRND = 8

_mesh = plsc.VectorSubcoreMesh(core_axis_name="c", subcore_axis_name="s")
_sc_params = pltpu.CompilerParams(needs_layout_passes=False)


def _wid():
    return lax.axis_index("s") * 2 + lax.axis_index("c")


@functools.partial(
    pl.kernel,
    mesh=_mesh,
    out_type=(
        jax.ShapeDtypeStruct((NP,), jnp.float32),
        jax.ShapeDtypeStruct((NP,), jnp.float32),
    ),
    scratch_types=[
        pltpu.VMEM((EC,), jnp.int32),
        pltpu.VMEM((EC,), jnp.int32),
        pltpu.VMEM((NPT,), jnp.float32),
        pltpu.VMEM((NPT,), jnp.float32),
    ],
    compiler_params=_sc_params,
)
def _degrees(src_hbm, dst_hbm, dego_hbm, degi_hbm, sbuf, dbuf, acco, acci):
    wid = _wid()
    lo = wid * NPT
    hi = lo + NPT
    zeros = jnp.zeros((16,), jnp.float32)
    ones = jnp.ones((16,), jnp.float32)
    for j in range(NPT // 16):
        acco[pl.ds(j * 16, 16)] = zeros
        acci[pl.ds(j * 16, 16)] = zeros

    def chunk_body(ec, _):
        off = pl.multiple_of(ec * EC, 8)
        pltpu.sync_copy(src_hbm.at[pl.ds(off, EC)], sbuf)
        pltpu.sync_copy(dst_hbm.at[pl.ds(off, EC)], dbuf)

        def grp(i, _):
            svec = sbuf[pl.ds(i * 16, 16)]
            dvec = dbuf[pl.ds(i * 16, 16)]
            ms = (svec >= lo) & (svec < hi)
            md = (dvec >= lo) & (dvec < hi)
            si = jnp.clip(svec - lo, 0, NPT - 1)
            di = jnp.clip(dvec - lo, 0, NPT - 1)
            plsc.addupdate_scatter(acco, [si], ones, mask=ms)
            plsc.addupdate_scatter(acci, [di], ones, mask=md)
            return 0

        return lax.fori_loop(0, EC // 16, grp, 0)

    lax.fori_loop(0, NCH, chunk_body, 0)
    pltpu.sync_copy(acco, dego_hbm.at[pl.ds(lo, NPT)])
    pltpu.sync_copy(acci, degi_hbm.at[pl.ds(lo, NPT)])


@functools.partial(
    pl.kernel,
    mesh=_mesh,
    out_type=(
        jax.ShapeDtypeStruct((NW, CAP), jnp.int32),
        jax.ShapeDtypeStruct((NW, CAP), jnp.int32),
        jax.ShapeDtypeStruct((NW * 16,), jnp.int32),
    ),
    scratch_types=[
        pltpu.VMEM((EC,), jnp.int32),
        pltpu.VMEM((EC,), jnp.int32),
        pltpu.VMEM((2 * BLK + 16,), jnp.int32),
        pltpu.VMEM((2 * BLK + 16,), jnp.int32),
        pltpu.VMEM((16,), jnp.int32),
    ],
    compiler_params=_sc_params,
)
def _bucket(src_hbm, dst_hbm, bsrc_hbm, bldst_hbm, cnt_hbm,
            sbuf, dbuf, psrc, pdst, stg):
    wid = _wid()
    lo = pl.multiple_of(wid * NPT, 8)
    hi = lo + NPT

    def chunk_body(ec, carry):
        off = pl.multiple_of(ec * EC, 8)
        pltpu.sync_copy(src_hbm.at[pl.ds(off, EC)], sbuf)
        pltpu.sync_copy(dst_hbm.at[pl.ds(off, EC)], dbuf)

        def grp(i, ptr):
            for u in range(5):
                off16 = (i * 5 + u) * 16
                svec = sbuf[pl.ds(off16, 16)]
                dvec = dbuf[pl.ds(off16, 16)]
                m = (dvec >= lo) & (dvec < hi)
                plsc.store_compressed(psrc.at[pl.ds(ptr, 16)], svec, mask=m)
                plsc.store_compressed(pdst.at[pl.ds(ptr, 16)], dvec - lo,
                                      mask=m)
                ptr = ptr + plsc.all_reduce_population_count(m)[0]
            return ptr

        ptr = lax.fori_loop(0, EC // 80, grp, carry[0])

        def do_flush(c):
            p, b = c
            boff = pl.multiple_of(b * BLK, 8)
            pltpu.sync_copy(psrc.at[pl.ds(0, BLK)],
                            bsrc_hbm.at[wid, pl.ds(boff, BLK)])
            pltpu.sync_copy(pdst.at[pl.ds(0, BLK)],
                            bldst_hbm.at[wid, pl.ds(boff, BLK)])
            for j in range(BLK // 16):
                psrc[pl.ds(j * 16, 16)] = psrc[pl.ds(BLK + j * 16, 16)]
                pdst[pl.ds(j * 16, 16)] = pdst[pl.ds(BLK + j * 16, 16)]
            return (p - BLK, b + 1)

        return lax.cond(ptr >= BLK, do_flush, lambda c: c, (ptr, carry[1]))

    ptr, blk = lax.fori_loop(0, NCH, chunk_body,
                             (jnp.int32(0), jnp.int32(0)))
    psrc[pl.ds(ptr, 16)] = jnp.zeros((16,), jnp.int32)
    pdst[pl.ds(ptr, 16)] = jnp.full((16,), NPT, jnp.int32)
    boff = pl.multiple_of(blk * BLK, 8)
    pltpu.sync_copy(psrc.at[pl.ds(0, BLK)], bsrc_hbm.at[wid, pl.ds(boff, BLK)])
    pltpu.sync_copy(pdst.at[pl.ds(0, BLK)], bldst_hbm.at[wid, pl.ds(boff, BLK)])
    stg[...] = jnp.broadcast_to(blk * BLK + ptr, (16,))
    pltpu.sync_copy(stg, cnt_hbm.at[pl.ds(pl.multiple_of(wid * 16, 8), 16)])


@functools.partial(
    pl.kernel,
    mesh=_mesh,
    out_type=jax.ShapeDtypeStruct((NP, D), jnp.float32),
    scratch_types=[
        pltpu.VMEM((BLK,), jnp.int32),
        pltpu.VMEM((BLK,), jnp.int32),
        pltpu.VMEM((BLK + 32,), jnp.int32),
        pltpu.VMEM((BLK + 32,), jnp.int32),
        pltpu.VMEM((NBUF, G, D), jnp.float32),
        pltpu.VMEM((SCH + 1, D), jnp.float32),
    ] + [pltpu.SemaphoreType.DMA] * NBUF,
    compiler_params=_sc_params,
)
def _scatter_rows(bsrc_hbm, bldst_hbm, cnt_hbm, z_hbm, agg_hbm,
                  lbs, lbd, psrc, pdst, gbuf, acc, *sems):
    wid = _wid()
    zeros = jnp.zeros((16,), jnp.float32)
    trash = jnp.full((16,), SCH, jnp.int32)
    zsrc = jnp.zeros((16,), jnp.int32)

    pltpu.sync_copy(cnt_hbm.at[pl.ds(pl.multiple_of(wid * 16, 8), 16)],
                    lbs.at[pl.ds(0, 16)])
    cnt = lbs[pl.ds(0, 16)][0]
    cnt16 = ((cnt + 15) // 16) * 16
    nblk = (cnt16 + BLK - 1) // BLK

    def round_body(rnd, _):
        rlo = rnd * SCH
        rhi = rlo + SCH

        def zrow(r, _):
            for k in range(D // 16):
                acc[r, pl.ds(k * 16, 16)] = zeros
            return 0

        lax.fori_loop(0, SCH + 1, zrow, 0)

        def blk_body(bi, _):
            boff = pl.multiple_of(bi * BLK, 8)
            pltpu.sync_copy(bsrc_hbm.at[wid, pl.ds(boff, BLK)], lbs)
            pltpu.sync_copy(bldst_hbm.at[wid, pl.ds(boff, BLK)], lbd)
            ngrp = jnp.minimum(BLK, cnt16 - bi * BLK) // 16

            def grp(i, ptr):
                svec = lbs[pl.ds(i * 16, 16)]
                dvec = lbd[pl.ds(i * 16, 16)]
                m = (dvec >= rlo) & (dvec < rhi)
                plsc.store_compressed(psrc.at[pl.ds(ptr, 16)], svec, mask=m)
                plsc.store_compressed(pdst.at[pl.ds(ptr, 16)], dvec - rlo,
                                      mask=m)
                return ptr + plsc.all_reduce_population_count(m)[0]

            ptr = lax.fori_loop(0, ngrp, grp, jnp.int32(0))
            psrc[pl.ds(ptr, 16)] = zsrc
            pdst[pl.ds(ptr, 16)] = trash
            nb = (ptr + G - 1) // G

            def fire(g, buf):
                goff = pl.multiple_of(g * G, 8)
                pltpu.async_copy(z_hbm.at[psrc.at[pl.ds(goff, G)]],
                                 gbuf.at[buf], sems[buf])

            def wait(g, buf):
                goff = pl.multiple_of(g * G, 8)
                pltpu.make_async_copy(z_hbm.at[psrc.at[pl.ds(goff, G)]],
                                      gbuf.at[buf], sems[buf]).wait()

            def accum(g, buf):
                goff = pl.multiple_of(g * G, 8)

                def rows4(jj, _):
                    dvq = pdst[pl.ds(goff + jj * 4, 16)]
                    for q in range(4):
                        d = dvq[q]
                        gg = jj * 4 + q
                        for k in range(D // 16):
                            plsc.addupdate(
                                acc.at[d, pl.ds(k * 16, 16)],
                                gbuf[buf, gg, pl.ds(k * 16, 16)],
                            )
                    return 0

                lax.fori_loop(0, G // 4, rows4, 0)

            for u in range(NBUF):
                @pl.when(u < nb)
                def _(u=u):
                    fire(u, u)

            def drainN(j, _):
                g0 = j * NBUF
                for u in range(NBUF):
                    @pl.when(g0 + u < nb)
                    def _(u=u):
                        wait(g0 + u, u)
                        accum(g0 + u, u)

                        @pl.when(g0 + u + NBUF < nb)
                        def _(u=u):
                            fire(g0 + u + NBUF, u)

                return 0

            lax.fori_loop(0, (nb + NBUF - 1) // NBUF, drainN, 0)
            return 0

        lax.fori_loop(0, nblk, blk_body, 0)
        out_lo = pl.multiple_of(wid * NPT + rlo, 8)
        pltpu.sync_copy(acc.at[pl.ds(0, SCH)], agg_hbm.at[pl.ds(out_lo, SCH)])
        return 0

    lax.fori_loop(0, NRND, round_body, 0)


TN = 512


def _norm(deg):
    return lax.rsqrt(jnp.maximum(deg, 1.0))


def _l1_body(x_ref, dego_ref, w1_ref, z_ref):
    xb = x_ref[0]
    ns = _norm(dego_ref[...])
    y = lax.dot_general(xb, w1_ref[...], (((0,), (0,)), ((), ())),
                        preferred_element_type=jnp.float32)
    z_ref[...] = y * ns[:, None]


def _layer1_matmul(x_p, deg_out, W1):
    return pl.pallas_call(
        _l1_body,
        grid=(NP // TN, FB),
        in_specs=[
            pl.BlockSpec((1, FC, TN), lambda n, b: (b, 0, n)),
            pl.BlockSpec((TN,), lambda n, b: (n,)),
            pl.BlockSpec((FC, FC), lambda n, b: (0, 0)),
        ],
        out_specs=pl.BlockSpec((TN, FC), lambda n, b: (n, b)),
        out_shape=jax.ShapeDtypeStruct((NP, D), jnp.float32),
    )(x_p, deg_out, W1)


def _l2_body(agg_ref, dego_ref, degi_ref, b1_ref, w2_ref, z_ref):
    ns = _norm(dego_ref[...])
    nd = _norm(degi_ref[...])
    h = jnp.maximum(agg_ref[...] * nd[:, None] + b1_ref[...][None, :], 0.0)
    z_ref[...] = lax.dot_general(
        h * ns[:, None], w2_ref[...], (((1,), (0,)), ((), ())),
        preferred_element_type=jnp.float32)


def _layer2_matmul(agg1, deg_out, deg_in, b1, W2):
    return pl.pallas_call(
        _l2_body,
        grid=(NP // TN, FB),
        in_specs=[
            pl.BlockSpec((TN, FC), lambda n, b: (n, b)),
            pl.BlockSpec((TN,), lambda n, b: (n,)),
            pl.BlockSpec((TN,), lambda n, b: (n,)),
            pl.BlockSpec((FC,), lambda n, b: (0,)),
            pl.BlockSpec((FC, FC), lambda n, b: (0, 0)),
        ],
        out_specs=pl.BlockSpec((TN, FC), lambda n, b: (n, b)),
        out_shape=jax.ShapeDtypeStruct((NP, D), jnp.float32),
    )(agg1, deg_out, deg_in, b1, W2)


def _fin_body(agg_ref, degi_ref, b2_ref, o_ref):
    nd = _norm(degi_ref[...])
    o_ref[...] = agg_ref[...] * nd[:, None] + b2_ref[...][None, :]


def _finalize(agg2, deg_in, b2):
    return pl.pallas_call(
        _fin_body,
        grid=(NP // TN, FB),
        in_specs=[
            pl.BlockSpec((TN, FC), lambda n, b: (n, b)),
            pl.BlockSpec((TN,), lambda n, b: (n,)),
            pl.BlockSpec((FC,), lambda n, b: (0,)),
        ],
        out_specs=pl.BlockSpec((TN, FC), lambda n, b: (n, b)),
        out_shape=jax.ShapeDtypeStruct((NP, D), jnp.float32),
    )(agg2, deg_in, b2)


def kernel(inputs, edge_index, W1, b1, W2, b2):
    b, c, h, w = inputs.shape
    x = inputs.reshape(b, c, h * w)
    x_p = jnp.pad(x, ((0, 0), (0, 0), (0, NP - N)))
    src = edge_index[0]
    dst = edge_index[1]

    deg_out, deg_in = _degrees(src, dst)
    bsrc, bldst, bcnt = _bucket(src, dst)
    z1 = _layer1_matmul(x_p, deg_out, W1)
    agg1 = _scatter_rows(bsrc, bldst, bcnt, z1)
    z2 = _layer2_matmul(agg1, deg_out, deg_in, b1, W2)
    agg2 = _scatter_rows(bsrc, bldst, bcnt, z2)
    onb = _finalize(agg2, deg_in, b2)

    out = onb[:N].reshape(N, b, c).transpose(1, 2, 0)
    return out.reshape(b, c, h, w)


# speedup vs baseline: 1.2359x; 1.1666x over previous
import functools

import jax
import jax.numpy as jnp
from jax import lax
from jax.experimental import pallas as pl
from jax.experimental.pallas import tpu as pltpu
from jax.experimental.pallas import tpu_sc as plsc

N = 10000
NP = 10240
E = 160000
FB, FC = 2, 256
D = FB * FC

NW = 32
EC = 2000
NCH = E // EC
G = 16
NBUF = 9

NPT = NP // NW
BLK = 2048
CAP = 80 * BLK
SCH = 80
NRND = 4

_mesh = plsc.VectorSubcoreMesh(core_axis_name="c", subcore_axis_name="s")
_sc_params = pltpu.CompilerParams(needs_layout_passes=False)


def _wid():
    return lax.axis_index("s") * 2 + lax.axis_index("c")


@functools.partial(
    pl.kernel,
    mesh=_mesh,
    out_type=(
        jax.ShapeDtypeStruct((NW, CAP), jnp.int32),
        jax.ShapeDtypeStruct((NW, CAP), jnp.int32),
        jax.ShapeDtypeStruct((NW * 16,), jnp.int32),
        jax.ShapeDtypeStruct((NP,), jnp.float32),
        jax.ShapeDtypeStruct((NP,), jnp.float32),
    ),
    scratch_types=[
        pltpu.VMEM((EC,), jnp.int32),
        pltpu.VMEM((EC,), jnp.int32),
        pltpu.VMEM((2 * BLK + 16,), jnp.int32),
        pltpu.VMEM((2 * BLK + 16,), jnp.int32),
        pltpu.VMEM((16,), jnp.int32),
        pltpu.VMEM((NPT,), jnp.float32),
        pltpu.VMEM((NPT,), jnp.float32),
    ],
    compiler_params=_sc_params,
)
def _bucket(src_hbm, dst_hbm, bsrc_hbm, bldst_hbm, cnt_hbm,
            dego_hbm, degi_hbm, sbuf, dbuf, psrc, pdst, stg, acco, acci):
    wid = _wid()
    lo = pl.multiple_of(wid * NPT, 8)
    hi = lo + NPT
    zf = jnp.zeros((16,), jnp.float32)
    ones = jnp.ones((16,), jnp.float32)
    for j in range(NPT // 16):
        acco[pl.ds(j * 16, 16)] = zf
        acci[pl.ds(j * 16, 16)] = zf

    def chunk_body(ec, carry):
        off = pl.multiple_of(ec * EC, 8)
        pltpu.sync_copy(src_hbm.at[pl.ds(off, EC)], sbuf)
        pltpu.sync_copy(dst_hbm.at[pl.ds(off, EC)], dbuf)

        def grp(i, ptr):
            for u in range(5):
                off16 = (i * 5 + u) * 16
                svec = sbuf[pl.ds(off16, 16)]
                dvec = dbuf[pl.ds(off16, 16)]
                m = (dvec >= lo) & (dvec < hi)
                plsc.store_compressed(psrc.at[pl.ds(ptr, 16)], svec, mask=m)
                plsc.store_compressed(pdst.at[pl.ds(ptr, 16)], dvec - lo,
                                      mask=m)
                ptr = ptr + plsc.all_reduce_population_count(m)[0]
                ms = (svec >= lo) & (svec < hi)
                si = jnp.clip(svec - lo, 0, NPT - 1)
                di = jnp.clip(dvec - lo, 0, NPT - 1)
                plsc.addupdate_scatter(acco, [si], ones, mask=ms)
                plsc.addupdate_scatter(acci, [di], ones, mask=m)
            return ptr

        ptr = lax.fori_loop(0, EC // 80, grp, carry[0])

        def do_flush(c):
            p, b = c
            boff = pl.multiple_of(b * BLK, 8)
            pltpu.sync_copy(psrc.at[pl.ds(0, BLK)],
                            bsrc_hbm.at[wid, pl.ds(boff, BLK)])
            pltpu.sync_copy(pdst.at[pl.ds(0, BLK)],
                            bldst_hbm.at[wid, pl.ds(boff, BLK)])
            for j in range(BLK // 16):
                psrc[pl.ds(j * 16, 16)] = psrc[pl.ds(BLK + j * 16, 16)]
                pdst[pl.ds(j * 16, 16)] = pdst[pl.ds(BLK + j * 16, 16)]
            return (p - BLK, b + 1)

        return lax.cond(ptr >= BLK, do_flush, lambda c: c, (ptr, carry[1]))

    ptr, blk = lax.fori_loop(0, NCH, chunk_body,
                             (jnp.int32(0), jnp.int32(0)))
    psrc[pl.ds(ptr, 16)] = jnp.zeros((16,), jnp.int32)
    pdst[pl.ds(ptr, 16)] = jnp.full((16,), NPT, jnp.int32)
    boff = pl.multiple_of(blk * BLK, 8)
    pltpu.sync_copy(psrc.at[pl.ds(0, BLK)], bsrc_hbm.at[wid, pl.ds(boff, BLK)])
    pltpu.sync_copy(pdst.at[pl.ds(0, BLK)], bldst_hbm.at[wid, pl.ds(boff, BLK)])
    stg[...] = jnp.broadcast_to(blk * BLK + ptr, (16,))
    pltpu.sync_copy(stg, cnt_hbm.at[pl.ds(pl.multiple_of(wid * 16, 8), 16)])
    pltpu.sync_copy(acco, dego_hbm.at[pl.ds(lo, NPT)])
    pltpu.sync_copy(acci, degi_hbm.at[pl.ds(lo, NPT)])


@functools.partial(
    pl.kernel,
    mesh=_mesh,
    out_type=jax.ShapeDtypeStruct((NP, D), jnp.float32),
    scratch_types=[
        pltpu.VMEM((BLK,), jnp.int32),
        pltpu.VMEM((BLK,), jnp.int32),
        pltpu.VMEM((BLK + 32,), jnp.int32),
        pltpu.VMEM((BLK + 32,), jnp.int32),
        pltpu.VMEM((NBUF, G, D), jnp.float32),
        pltpu.VMEM((SCH + 1, D), jnp.float32),
    ] + [pltpu.SemaphoreType.DMA] * NBUF,
    compiler_params=_sc_params,
)
def _scatter_rows(bsrc_hbm, bldst_hbm, cnt_hbm, z_hbm, agg_hbm,
                  lbs, lbd, psrc, pdst, gbuf, acc, *sems):
    wid = _wid()
    zeros = jnp.zeros((16,), jnp.float32)
    trash = jnp.full((16,), SCH, jnp.int32)
    zsrc = jnp.zeros((16,), jnp.int32)

    pltpu.sync_copy(cnt_hbm.at[pl.ds(pl.multiple_of(wid * 16, 8), 16)],
                    lbs.at[pl.ds(0, 16)])
    cnt = lbs[pl.ds(0, 16)][0]
    cnt16 = ((cnt + 15) // 16) * 16
    nblk = (cnt16 + BLK - 1) // BLK

    def round_body(rnd, _):
        rlo = rnd * SCH
        rhi = rlo + SCH

        def zrow(r, _):
            for k in range(D // 16):
                acc[r, pl.ds(k * 16, 16)] = zeros
            return 0

        lax.fori_loop(0, SCH + 1, zrow, 0)

        def blk_body(bi, _):
            boff = pl.multiple_of(bi * BLK, 8)
            pltpu.sync_copy(bsrc_hbm.at[wid, pl.ds(boff, BLK)], lbs)
            pltpu.sync_copy(bldst_hbm.at[wid, pl.ds(boff, BLK)], lbd)
            ngrp = jnp.minimum(BLK, cnt16 - bi * BLK) // 16

            def grp(i, ptr):
                svec = lbs[pl.ds(i * 16, 16)]
                dvec = lbd[pl.ds(i * 16, 16)]
                m = (dvec >= rlo) & (dvec < rhi)
                plsc.store_compressed(psrc.at[pl.ds(ptr, 16)], svec, mask=m)
                plsc.store_compressed(pdst.at[pl.ds(ptr, 16)], dvec - rlo,
                                      mask=m)
                return ptr + plsc.all_reduce_population_count(m)[0]

            ptr = lax.fori_loop(0, ngrp, grp, jnp.int32(0))
            psrc[pl.ds(ptr, 16)] = zsrc
            pdst[pl.ds(ptr, 16)] = trash
            nb = (ptr + G - 1) // G

            def fire(g, buf):
                goff = pl.multiple_of(g * G, 8)
                pltpu.async_copy(z_hbm.at[psrc.at[pl.ds(goff, G)]],
                                 gbuf.at[buf], sems[buf])

            def wait(g, buf):
                goff = pl.multiple_of(g * G, 8)
                pltpu.make_async_copy(z_hbm.at[psrc.at[pl.ds(goff, G)]],
                                      gbuf.at[buf], sems[buf]).wait()

            def accum(g, buf):
                goff = pl.multiple_of(g * G, 8)

                def rows4(jj, _):
                    dvq = pdst[pl.ds(goff + jj * 4, 16)]
                    for q in range(4):
                        d = dvq[q]
                        gg = jj * 4 + q
                        for k in range(D // 16):
                            plsc.addupdate(
                                acc.at[d, pl.ds(k * 16, 16)],
                                gbuf[buf, gg, pl.ds(k * 16, 16)],
                            )
                    return 0

                lax.fori_loop(0, G // 4, rows4, 0)

            for u in range(NBUF):
                @pl.when(u < nb)
                def _(u=u):
                    fire(u, u)

            def drainN(j, _):
                g0 = j * NBUF
                for u in range(NBUF):
                    @pl.when(g0 + u < nb)
                    def _(u=u):
                        wait(g0 + u, u)
                        accum(g0 + u, u)

                        @pl.when(g0 + u + NBUF < nb)
                        def _(u=u):
                            fire(g0 + u + NBUF, u)

                return 0

            lax.fori_loop(0, (nb + NBUF - 1) // NBUF, drainN, 0)
            return 0

        lax.fori_loop(0, nblk, blk_body, 0)
        out_lo = pl.multiple_of(wid * NPT + rlo, 8)
        pltpu.sync_copy(acc.at[pl.ds(0, SCH)], agg_hbm.at[pl.ds(out_lo, SCH)])
        return 0

    lax.fori_loop(0, NRND, round_body, 0)


TN = 512


def _norm(deg):
    return lax.rsqrt(jnp.maximum(deg, 1.0))


def _l1_body(x_ref, dego_ref, w1_ref, z_ref):
    xb = x_ref[0]
    ns = _norm(dego_ref[...])
    y = lax.dot_general(xb, w1_ref[...], (((0,), (0,)), ((), ())),
                        preferred_element_type=jnp.float32)
    z_ref[...] = y * ns[:, None]


def _layer1_matmul(x_p, deg_out, W1):
    return pl.pallas_call(
        _l1_body,
        grid=(NP // TN, FB),
        in_specs=[
            pl.BlockSpec((1, FC, TN), lambda n, b: (b, 0, n)),
            pl.BlockSpec((TN,), lambda n, b: (n,)),
            pl.BlockSpec((FC, FC), lambda n, b: (0, 0)),
        ],
        out_specs=pl.BlockSpec((TN, FC), lambda n, b: (n, b)),
        out_shape=jax.ShapeDtypeStruct((NP, D), jnp.float32),
    )(x_p, deg_out, W1)


def _l2_body(agg_ref, dego_ref, degi_ref, b1_ref, w2_ref, z_ref):
    ns = _norm(dego_ref[...])
    nd = _norm(degi_ref[...])
    h = jnp.maximum(agg_ref[...] * nd[:, None] + b1_ref[...][None, :], 0.0)
    z_ref[...] = lax.dot_general(
        h * ns[:, None], w2_ref[...], (((1,), (0,)), ((), ())),
        preferred_element_type=jnp.float32)


def _layer2_matmul(agg1, deg_out, deg_in, b1, W2):
    return pl.pallas_call(
        _l2_body,
        grid=(NP // TN, FB),
        in_specs=[
            pl.BlockSpec((TN, FC), lambda n, b: (n, b)),
            pl.BlockSpec((TN,), lambda n, b: (n,)),
            pl.BlockSpec((TN,), lambda n, b: (n,)),
            pl.BlockSpec((FC,), lambda n, b: (0,)),
            pl.BlockSpec((FC, FC), lambda n, b: (0, 0)),
        ],
        out_specs=pl.BlockSpec((TN, FC), lambda n, b: (n, b)),
        out_shape=jax.ShapeDtypeStruct((NP, D), jnp.float32),
    )(agg1, deg_out, deg_in, b1, W2)


def _fin_body(agg_ref, degi_ref, b2_ref, o_ref):
    nd = _norm(degi_ref[...])
    o_ref[...] = agg_ref[...] * nd[:, None] + b2_ref[...][None, :]


def _finalize(agg2, deg_in, b2):
    return pl.pallas_call(
        _fin_body,
        grid=(NP // TN, FB),
        in_specs=[
            pl.BlockSpec((TN, FC), lambda n, b: (n, b)),
            pl.BlockSpec((TN,), lambda n, b: (n,)),
            pl.BlockSpec((FC,), lambda n, b: (0,)),
        ],
        out_specs=pl.BlockSpec((TN, FC), lambda n, b: (n, b)),
        out_shape=jax.ShapeDtypeStruct((NP, D), jnp.float32),
    )(agg2, deg_in, b2)


def kernel(inputs, edge_index, W1, b1, W2, b2):
    b, c, h, w = inputs.shape
    x = inputs.reshape(b, c, h * w)
    x_p = jnp.pad(x, ((0, 0), (0, 0), (0, NP - N)))
    src = edge_index[0]
    dst = edge_index[1]

    bsrc, bldst, bcnt, deg_out, deg_in = _bucket(src, dst)
    z1 = _layer1_matmul(x_p, deg_out, W1)
    agg1 = _scatter_rows(bsrc, bldst, bcnt, z1)
    z2 = _layer2_matmul(agg1, deg_out, deg_in, b1, W2)
    agg2 = _scatter_rows(bsrc, bldst, bcnt, z2)
    onb = _finalize(agg2, deg_in, b2)

    out = onb[:N].reshape(N, b, c).transpose(1, 2, 0)
    return out.reshape(b, c, h, w)
